# baseline (speedup 1.0000x reference)
import functools

import jax
import jax.numpy as jnp
from jax import lax
from jax.experimental import pallas as pl
from jax.experimental.pallas import tpu as pltpu
from jax.experimental.pallas import tpu_sc as plsc

VOCAB = 1000000
EMBED = 64
BATCH = 16384
NEG = 5
NIDX = NEG + 1
WIDE = 128

NC, NS, LANES = 2, 16, 16
NW = NC * NS
BPW = BATCH // NW
CB = 128
NCHUNK = BPW // CB
NGROUP = CB // LANES
SROWS = 8

LBLK = 896
NBLK = (VOCAB - EMBED) // LBLK
TAILV = VOCAB - NBLK * LBLK
_SC_PARAMS = pltpu.CompilerParams(needs_layout_passes=False)


def _sc_pack_tables(w_t, v_t, w_tail, v_tail):
    mesh = plsc.VectorSubcoreMesh(core_axis_name="c", subcore_axis_name="s")
    out_sds = jax.ShapeDtypeStruct((VOCAB // 2, WIDE), jnp.float32)

    @functools.partial(
        pl.kernel,
        out_type=(out_sds, out_sds),
        mesh=mesh,
        compiler_params=_SC_PARAMS,
        scratch_types=[
            pltpu.VMEM((EMBED, LBLK), jnp.float32),
            pltpu.VMEM((LBLK // 2, WIDE), jnp.float32),
            pltpu.VMEM((TAILV // 2, WIDE), jnp.float32),
        ],
    )
    def k(wt_hbm, vt_hbm, wtail_hbm, vtail_hbm, wout_hbm, vout_hbm,
          in_v, out_v, tail_v):
        wid = lax.axis_index("s") * NC + lax.axis_index("c")
        lane = lax.iota(jnp.int32, LANES)
        rowbase = lane >> 1
        colpar = (lane & 1) << 6
        nb = 34 + jnp.where(wid < NBLK - 34 * NW, 1, 0)

        for t_hbm, o_hbm in ((wt_hbm, wout_hbm), (vt_hbm, vout_hbm)):
            def blk(kk, _, t_hbm=t_hbm, o_hbm=o_hbm):
                b = wid + NW * kk
                off = pl.multiple_of(b * LBLK, 128)
                pltpu.sync_copy(t_hbm.at[:, pl.ds(off, LBLK)], in_v)

                def drow(d, _):
                    cols = colpar + jnp.full((LANES,), d, jnp.int32)
                    for gr in range(LBLK // LANES):
                        val = in_v[d, pl.ds(gr * LANES, LANES)]
                        plsc.store_scatter(out_v, [gr * 8 + rowbase, cols],
                                           val)
                    return 0

                lax.fori_loop(0, EMBED, drow, 0)
                ooff = pl.multiple_of(b * (LBLK // 2), 8)
                pltpu.sync_copy(out_v, o_hbm.at[pl.ds(ooff, LBLK // 2)])
                return 0

            lax.fori_loop(0, nb, blk, 0)

        @pl.when(wid == 5)
        def _():
            pltpu.sync_copy(wtail_hbm, tail_v)
            pltpu.sync_copy(tail_v,
                            wout_hbm.at[pl.ds(NBLK * LBLK // 2, TAILV // 2)])

        @pl.when(wid == 9)
        def _():
            pltpu.sync_copy(vtail_hbm, tail_v)
            pltpu.sync_copy(tail_v,
                            vout_hbm.at[pl.ds(NBLK * LBLK // 2, TAILV // 2)])

    return k(w_t, v_t, w_tail, v_tail)


def _sc_scores(pos_w, vidx, w_table, v_table):
    mesh = plsc.VectorSubcoreMesh(core_axis_name="c", subcore_axis_name="s")

    @functools.partial(
        pl.kernel,
        out_type=jax.ShapeDtypeStruct((NW, NCHUNK, SROWS, CB), jnp.float32),
        mesh=mesh,
        compiler_params=_SC_PARAMS,
        scratch_types=[
            pltpu.VMEM((BPW,), jnp.int32),
            pltpu.VMEM((NIDX, BPW), jnp.int32),
            pltpu.VMEM((CB,), jnp.int32),
            pltpu.VMEM((NIDX, CB), jnp.int32),
            pltpu.VMEM((CB, WIDE), jnp.float32),
            pltpu.VMEM((NIDX, CB, WIDE), jnp.float32),
            pltpu.VMEM((SROWS, CB), jnp.float32),
            pltpu.SemaphoreType.DMA,
        ],
    )
    def k(pos_w_hbm, vidx_hbm, w_hbm, v_hbm, out_hbm,
          widx_v, vidx_v, widx2_v, vidx2_v, wrows, vrows, scores_v, sem):
        wid = lax.axis_index("s") * NC + lax.axis_index("c")
        lane = lax.iota(jnp.int32, LANES)
        base = wid * BPW

        icps = [pltpu.async_copy(pos_w_hbm.at[pl.ds(base, BPW)], widx_v,
                                 sem)]
        for j in range(NIDX):
            icps.append(pltpu.async_copy(vidx_hbm.at[j, pl.ds(base, BPW)],
                                         vidx_v.at[j], sem))
        for cp in icps:
            cp.wait()

        zero = jnp.zeros((LANES,), jnp.float32)
        for r in range(NIDX, SROWS):
            for g in range(NGROUP):
                scores_v[r, pl.ds(g * LANES, LANES)] = zero

        for chunk in range(NCHUNK):
            coff = chunk * CB
            for g in range(NGROUP):
                sl = pl.ds(g * LANES, LANES)
                fl = pl.ds(coff + g * LANES, LANES)
                widx2_v[sl] = widx_v[fl] >> 1
                for j in range(NIDX):
                    vidx2_v[j, sl] = vidx_v[j, fl] >> 1
            cps = [pltpu.async_copy(w_hbm.at[widx2_v], wrows, sem)]
            for j in range(NIDX):
                cps.append(pltpu.async_copy(v_hbm.at[vidx2_v.at[j]],
                                            vrows.at[j], sem))
            for cp in cps:
                cp.wait()

            for g in range(NGROUP):
                sl = pl.ds(g * LANES, LANES)
                fl = pl.ds(coff + g * LANES, LANES)
                i_vec = jnp.full((LANES,), g * LANES, jnp.int32) + lane
                wo = (widx_v[fl] & 1) << 6
                vo = [(vidx_v[j, fl] & 1) << 6 for j in range(NIDX)]

                def body(d, accs, i_vec=i_vec, wo=wo, vo=vo):
                    d_vec = jnp.full((LANES,), d, jnp.int32)
                    wv = plsc.load_gather(wrows, [i_vec, wo + d_vec])
                    return tuple(
                        accs[j] + wv * plsc.load_gather(
                            vrows,
                            [jnp.full((LANES,), j, jnp.int32), i_vec,
                             vo[j] + d_vec])
                        for j in range(NIDX))

                accs = lax.fori_loop(0, EMBED, body, (zero,) * NIDX)
                scores_v[0, sl] = accs[0]
                for j in range(1, NIDX):
                    scores_v[j, sl] = -accs[j]

            pltpu.sync_copy(scores_v, out_hbm.at[wid, chunk])

    return k(pos_w, vidx, w_table, v_table)


def _tc_loss_body(x_ref, o_ref):
    x = jnp.clip(x_ref[...], -10.0, 10.0)
    row = lax.broadcasted_iota(jnp.int32, x.shape, 0)
    valid = (row % SROWS) < NIDX
    o_ref[0, 0] = -jnp.sum(jnp.where(valid, jax.nn.log_sigmoid(x), 0.0))


def kernel(pos_w, pos_v, neg_v, w_embeddings, v_embeddings):
    pos_w = jnp.asarray(pos_w, jnp.int32)
    vidx = jnp.concatenate(
        [jnp.asarray(pos_v, jnp.int32)[None, :],
         jnp.asarray(neg_v, jnp.int32).T], axis=0)

    w_tail = w_embeddings[NBLK * LBLK:].reshape(TAILV // 2, WIDE)
    v_tail = v_embeddings[NBLK * LBLK:].reshape(TAILV // 2, WIDE)
    w2, v2 = _sc_pack_tables(w_embeddings.T, v_embeddings.T, w_tail, v_tail)

    scores = _sc_scores(pos_w, vidx, w2, v2)
    flat = scores.reshape(NW * NCHUNK * SROWS, CB)

    loss = pl.pallas_call(
        _tc_loss_body,
        out_shape=jax.ShapeDtypeStruct((1, 1), jnp.float32),
        out_specs=pl.BlockSpec(memory_space=pltpu.SMEM),
    )(flat)
    return loss[0, 0]


# speedup vs baseline: 3.4686x; 3.4686x over previous
import functools

import jax
import jax.numpy as jnp
from jax import lax
from jax.experimental import pallas as pl
from jax.experimental.pallas import tpu as pltpu
from jax.experimental.pallas import tpu_sc as plsc

VOCAB = 1000000
EMBED = 64
BATCH = 16384
NEG = 5
NIDX = NEG + 1
NROW = NIDX + 1

NC, NS, LANES = 2, 16, 16
NW = NC * NS
BPW = BATCH // NW
CB = 64
NCHUNK = BPW // CB
NGROUP = CB // LANES


def _sc_scores(pos_w, vidx, w_table, v_table):
    mesh = plsc.VectorSubcoreMesh(core_axis_name="c", subcore_axis_name="s")

    @functools.partial(
        pl.kernel,
        out_type=jax.ShapeDtypeStruct((NW, NCHUNK, NIDX, CB), jnp.float32),
        mesh=mesh,
        compiler_params=pltpu.CompilerParams(needs_layout_passes=False),
        scratch_types=[
            pltpu.VMEM((BPW,), jnp.int32),
            pltpu.VMEM((NIDX, BPW), jnp.int32),
            pltpu.VMEM((2, CB, EMBED), jnp.float32),
            pltpu.VMEM((2, NIDX, CB, EMBED), jnp.float32),
            pltpu.VMEM((NIDX, CB), jnp.float32),
            pltpu.SemaphoreType.DMA,
            pltpu.SemaphoreType.DMA,
        ],
    )
    def k(pos_w_hbm, vidx_hbm, w_hbm, v_hbm, out_hbm,
          widx_v, vidx_v, wrows, vrows, scores_v, sem0, sem1):
        wid = lax.axis_index("s") * NC + lax.axis_index("c")
        lane = lax.iota(jnp.int32, LANES)
        zero = jnp.zeros((LANES,), jnp.float32)
        sems = (sem0, sem1)
        base = wid * BPW

        icps = [pltpu.async_copy(pos_w_hbm.at[pl.ds(base, BPW)], widx_v,
                                 sem0)]
        for j in range(NIDX):
            icps.append(pltpu.async_copy(vidx_hbm.at[j, pl.ds(base, BPW)],
                                         vidx_v.at[j], sem0))
        for cp in icps:
            cp.wait()

        def enqueue(c, buf):
            sem = sems[buf]

            def enq(g, _):
                off = c * CB + g * LANES
                wvec = widx_v[pl.ds(off, LANES)]
                vvecs = [vidx_v[j, pl.ds(off, LANES)] for j in range(NIDX)]
                for l in range(LANES):
                    i = g * LANES + l
                    pltpu.async_copy(w_hbm.at[wvec[l]], wrows.at[buf, i],
                                     sem)
                    for j in range(NIDX):
                        pltpu.async_copy(v_hbm.at[vvecs[j][l]],
                                         vrows.at[buf, j, i], sem)
                return 0

            lax.fori_loop(0, NGROUP, enq, 0)

        def drain(buf):
            def one(i, _):
                pltpu.make_async_copy(w_hbm.at[0], wrows.at[buf, 0],
                                      sems[buf]).wait()
                return 0

            lax.fori_loop(0, NROW * CB, one, 0)

        def compute(c, buf):
            for g in range(NGROUP):
                sl = pl.ds(g * LANES, LANES)
                i_vec = jnp.full((LANES,), g * LANES, jnp.int32) + lane
                b_vec = jnp.full((LANES,), buf, jnp.int32)

                def body(d, accs, i_vec=i_vec, b_vec=b_vec):
                    d_vec = jnp.full((LANES,), d, jnp.int32)
                    wv = plsc.load_gather(wrows, [b_vec, i_vec, d_vec])
                    return tuple(
                        accs[j] + wv * plsc.load_gather(
                            vrows,
                            [b_vec, jnp.full((LANES,), j, jnp.int32), i_vec,
                             d_vec])
                        for j in range(NIDX))

                accs = lax.fori_loop(0, EMBED, body, (zero,) * NIDX)
                scores_v[0, sl] = accs[0]
                for j in range(1, NIDX):
                    scores_v[j, sl] = -accs[j]

            pltpu.sync_copy(scores_v, out_hbm.at[wid, c])

        enqueue(0, 0)
        for c in range(NCHUNK):
            if c + 1 < NCHUNK:
                enqueue(c + 1, (c + 1) % 2)
            drain(c % 2)
            compute(c, c % 2)

    return k(pos_w, vidx, w_table, v_table)


def _tc_loss_body(x_ref, o_ref):
    x = jnp.clip(x_ref[...], -10.0, 10.0)
    o_ref[0, 0] = -jnp.sum(jax.nn.log_sigmoid(x))


def kernel(pos_w, pos_v, neg_v, w_embeddings, v_embeddings):
    pos_w = jnp.asarray(pos_w, jnp.int32)
    vidx = jnp.concatenate(
        [jnp.asarray(pos_v, jnp.int32)[None, :],
         jnp.asarray(neg_v, jnp.int32).T], axis=0)

    scores = _sc_scores(pos_w, vidx, w_embeddings, v_embeddings)
    flat = scores.reshape(BATCH * NIDX // 128, 128)

    loss = pl.pallas_call(
        _tc_loss_body,
        out_shape=jax.ShapeDtypeStruct((1, 1), jnp.float32),
        out_specs=pl.BlockSpec(memory_space=pltpu.SMEM),
    )(flat)
    return loss[0, 0]


# speedup vs baseline: 3.4881x; 1.0056x over previous
import functools

import jax
import jax.numpy as jnp
from jax import lax
from jax.experimental import pallas as pl
from jax.experimental.pallas import tpu as pltpu
from jax.experimental.pallas import tpu_sc as plsc

VOCAB = 1000000
EMBED = 64
BATCH = 16384
NEG = 5
NIDX = NEG + 1

NC, NS, LANES = 2, 16, 16
NW = NC * NS
BPW = BATCH // NW
CB = 64
NCHUNK = BPW // CB
NGROUP = CB // LANES
CSZ = NIDX * CB * EMBED
_SC_PARAMS = pltpu.CompilerParams(needs_layout_passes=False)
_MESH = dict(core_axis_name="c", subcore_axis_name="s")


def _sc_gather_v(vidx, v_table):
    @functools.partial(
        pl.kernel,
        out_type=jax.ShapeDtypeStruct((NW * NCHUNK * CSZ,), jnp.float32),
        mesh=plsc.VectorSubcoreMesh(**_MESH),
        compiler_params=_SC_PARAMS,
        scratch_types=[
            pltpu.VMEM((NIDX, BPW), jnp.int32),
            pltpu.VMEM((2, NIDX, CB, EMBED), jnp.float32),
            pltpu.VMEM((CSZ,), jnp.float32),
            pltpu.SemaphoreType.DMA,
            pltpu.SemaphoreType.DMA,
        ],
    )
    def k(vidx_hbm, v_hbm, out_hbm, vidx_v, vrows, flat_v, sem0, sem1):
        wid = lax.axis_index("s") * NC + lax.axis_index("c")
        sems = (sem0, sem1)
        base = wid * BPW

        icps = [pltpu.async_copy(vidx_hbm.at[j, pl.ds(base, BPW)],
                                 vidx_v.at[j], sem0) for j in range(NIDX)]
        for cp in icps:
            cp.wait()

        def enqueue(c, p):
            sem = sems[p]

            def enq(g, _):
                vvecs = [vidx_v[j, pl.ds(c * CB + g * LANES, LANES)]
                         for j in range(NIDX)]
                for l in range(LANES):
                    i = g * LANES + l
                    for j in range(NIDX):
                        pltpu.async_copy(v_hbm.at[vvecs[j][l]],
                                         vrows.at[p, j, i], sem)
                return 0

            lax.fori_loop(0, NGROUP, enq, 0)

        def flush(c, p):
            sem = sems[p]

            def one(i, _):
                pltpu.make_async_copy(v_hbm.at[0], vrows.at[0, 0, 0],
                                      sem).wait()
                return 0

            lax.fori_loop(0, NIDX * CB, one, 0)

            p_vec = jnp.full((LANES,), p, jnp.int32)

            def repack(kk, _):
                jj = kk // CB
                ii = kk % CB
                for dq in range(EMBED // LANES):
                    flat_v[pl.ds(kk * EMBED + dq * LANES, LANES)] = (
                        vrows[p, jj, ii, pl.ds(dq * LANES, LANES)])
                return 0

            lax.fori_loop(0, NIDX * CB, repack, 0)
            pltpu.sync_copy(
                flat_v, out_hbm.at[pl.ds((wid * NCHUNK + c) * CSZ, CSZ)])

        enqueue(0, 0)
        for c in range(NCHUNK):
            if c + 1 < NCHUNK:
                enqueue(c + 1, (c + 1) % 2)
            flush(c, c % 2)

    return k(vidx, v_table)


def _sc_scores_w(pos_w, v_emb, w_table):
    @functools.partial(
        pl.kernel,
        out_type=jax.ShapeDtypeStruct((NW, NCHUNK, NIDX, CB), jnp.float32),
        mesh=plsc.VectorSubcoreMesh(**_MESH),
        compiler_params=_SC_PARAMS,
        scratch_types=[
            pltpu.VMEM((BPW,), jnp.int32),
            pltpu.VMEM((2, CB, EMBED), jnp.float32),
            pltpu.VMEM((CSZ,), jnp.float32),
            pltpu.VMEM((CSZ,), jnp.float32),
            pltpu.VMEM((NIDX, CB), jnp.float32),
            pltpu.SemaphoreType.DMA,
            pltpu.SemaphoreType.DMA,
        ],
    )
    def k(pos_w_hbm, vemb_hbm, w_hbm, out_hbm,
          widx_v, wrows, vb0, vb1, scores_v, sem0, sem1):
        wid = lax.axis_index("s") * NC + lax.axis_index("c")
        lane = lax.iota(jnp.int32, LANES)
        zero = jnp.zeros((LANES,), jnp.float32)
        vbufs, sems = (vb0, vb1), (sem0, sem1)
        base = wid * BPW

        pltpu.async_copy(pos_w_hbm.at[pl.ds(base, BPW)], widx_v,
                         sem0).wait()

        def enqueue(c, p):
            sem = sems[p]
            pltpu.async_copy(
                vemb_hbm.at[pl.ds((wid * NCHUNK + c) * CSZ, CSZ)],
                vbufs[p], sem)

            def enq(g, _):
                wvec = widx_v[pl.ds(c * CB + g * LANES, LANES)]
                for l in range(LANES):
                    pltpu.async_copy(w_hbm.at[wvec[l]],
                                     wrows.at[p, g * LANES + l], sem)
                return 0

            lax.fori_loop(0, NGROUP, enq, 0)

        def drain(p):
            sem = sems[p]
            pltpu.make_async_copy(vemb_hbm.at[pl.ds(0, CSZ)], vbufs[p],
                                  sem).wait()

            def one(i, _):
                pltpu.make_async_copy(w_hbm.at[0], wrows.at[0, 0],
                                      sem).wait()
                return 0

            lax.fori_loop(0, CB, one, 0)

        def compute(c, p):
            vb = vbufs[p]
            for g in range(NGROUP):
                sl = pl.ds(g * LANES, LANES)
                i_vec = jnp.full((LANES,), g * LANES, jnp.int32) + lane
                p_vec = jnp.full((LANES,), p, jnp.int32)
                i64 = i_vec * EMBED
                vbase = [jnp.full((LANES,), j * CB * EMBED, jnp.int32) + i64
                         for j in range(NIDX)]

                def body(d, accs, i_vec=i_vec, p_vec=p_vec, vbase=vbase):
                    d_vec = jnp.full((LANES,), d, jnp.int32)
                    wv = plsc.load_gather(wrows, [p_vec, i_vec, d_vec])
                    return tuple(
                        accs[j] + wv * plsc.load_gather(vb,
                                                        [vbase[j] + d_vec])
                        for j in range(NIDX))

                accs = lax.fori_loop(0, EMBED, body, (zero,) * NIDX)
                scores_v[0, sl] = accs[0]
                for j in range(1, NIDX):
                    scores_v[j, sl] = -accs[j]

            pltpu.sync_copy(scores_v, out_hbm.at[wid, c])

        enqueue(0, 0)
        for c in range(NCHUNK):
            if c + 1 < NCHUNK:
                enqueue(c + 1, (c + 1) % 2)
            drain(c % 2)
            compute(c, c % 2)

    return k(pos_w, v_emb, w_table)


def _tc_loss_body(x_ref, o_ref):
    x = jnp.clip(x_ref[...], -10.0, 10.0)
    o_ref[0, 0] = -jnp.sum(jax.nn.log_sigmoid(x))


def kernel(pos_w, pos_v, neg_v, w_embeddings, v_embeddings):
    pos_w = jnp.asarray(pos_w, jnp.int32)
    vidx = jnp.concatenate(
        [jnp.asarray(pos_v, jnp.int32)[None, :],
         jnp.asarray(neg_v, jnp.int32).T], axis=0)

    v_emb = _sc_gather_v(vidx, v_embeddings)
    scores = _sc_scores_w(pos_w, v_emb, w_embeddings)
    flat = scores.reshape(BATCH * NIDX // 128, 128)

    loss = pl.pallas_call(
        _tc_loss_body,
        out_shape=jax.ShapeDtypeStruct((1, 1), jnp.float32),
        out_specs=pl.BlockSpec(memory_space=pltpu.SMEM),
    )(flat)
    return loss[0, 0]
